# baseline (speedup 1.0000x reference)
import jax
import jax.numpy as jnp
from jax import lax
from jax.experimental import pallas as pl
from jax.experimental.pallas import tpu as pltpu
from jax.experimental.pallas import tpu_sc as plsc

_EMBED_DIM = 1152
_NUM_CORES = 2
_NUM_SUBCORES = 16
_NUM_WORKERS = _NUM_CORES * _NUM_SUBCORES
_T_MAIN = 48


def _gather_body(idx_hbm, table_hbm, out_hbm, tail_hbm,
                 idx0, idx1, rows0, rows1, tail0, tail1,
                 isem0, isem1, gsem0, gsem1, ssem0, ssem1, tsem0, tsem1):
    wid = lax.axis_index("s") * _NUM_CORES + lax.axis_index("c")
    n_b = idx_hbm.shape[0] // _NUM_WORKERS
    t = idx_hbm.shape[1]
    n_tail = t - _T_MAIN
    base = wid * n_b
    idxs = (idx0, idx1)
    bufs = (rows0, rows1)
    tails = (tail0, tail1)
    isems = (isem0, isem1)
    gsems = (gsem0, gsem1)
    ssems = (ssem0, ssem1)
    tsems = (tsem0, tsem1)

    def start_gathers(r, slot):
        pltpu.async_copy(
            table_hbm.at[idxs[slot].at[pl.ds(0, _T_MAIN)]],
            bufs[slot], gsems[slot])
        pltpu.async_copy(
            table_hbm.at[idxs[slot].at[pl.ds(_T_MAIN, 8)]],
            tails[slot], gsems[slot])

    def wait_gathers(r, slot):
        pltpu.make_async_copy(
            table_hbm.at[idxs[slot].at[pl.ds(0, _T_MAIN)]],
            bufs[slot], gsems[slot]).wait()
        pltpu.make_async_copy(
            table_hbm.at[idxs[slot].at[pl.ds(_T_MAIN, 8)]],
            tails[slot], gsems[slot]).wait()

    def start_stores(r, slot):
        pltpu.async_copy(
            bufs[slot], out_hbm.at[base + r, pl.ds(0, _T_MAIN)], ssems[slot])
        pltpu.async_copy(tails[slot], tail_hbm.at[base + r], tsems[slot])

    def wait_stores(r, slot):
        pltpu.make_async_copy(
            bufs[slot], out_hbm.at[base + r, pl.ds(0, _T_MAIN)],
            ssems[slot]).wait()
        pltpu.make_async_copy(
            tails[slot], tail_hbm.at[base + r], tsems[slot]).wait()

    pltpu.async_copy(idx_hbm.at[base], idx0, isem0)
    pltpu.async_copy(idx_hbm.at[base + 1], idx1, isem1)
    pltpu.make_async_copy(idx_hbm.at[base], idx0, isem0).wait()
    start_gathers(0, 0)

    def body(i, carry):
        for s in range(2):
            g = 2 * i + s
            ns = 1 - s
            wait_gathers(g, s)

            @pl.when(g + 2 < n_b)
            def _():
                pltpu.async_copy(idx_hbm.at[base + g + 2], idxs[s], isems[s])

            start_stores(g, s)

            @pl.when(g > 0)
            def _():
                wait_stores(g - 1, ns)

            @pl.when(g + 1 < n_b)
            def _():
                pltpu.make_async_copy(
                    idx_hbm.at[base + g + 1], idxs[ns], isems[ns]).wait()
                start_gathers(g + 1, ns)
        return carry

    lax.fori_loop(0, n_b // 2, body, 0)
    wait_stores(n_b - 1, 1)


@jax.jit
def kernel(input, weight):
    b, t = input.shape
    n_tail = t - _T_MAIN
    idx = jnp.pad(input.astype(jnp.int32), ((0, 0), (0, 56 - t)))
    mesh = plsc.VectorSubcoreMesh(core_axis_name="c", subcore_axis_name="s")
    out_main, out_tail = pl.kernel(
        _gather_body,
        out_type=(
            jax.ShapeDtypeStruct((b, t, _EMBED_DIM), jnp.float32),
            jax.ShapeDtypeStruct((b, 8, _EMBED_DIM), jnp.float32),
        ),
        mesh=mesh,
        scratch_types=[
            pltpu.VMEM((56,), jnp.int32),
            pltpu.VMEM((56,), jnp.int32),
            pltpu.VMEM((_T_MAIN, _EMBED_DIM), jnp.float32),
            pltpu.VMEM((_T_MAIN, _EMBED_DIM), jnp.float32),
            pltpu.VMEM((8, _EMBED_DIM), jnp.float32),
            pltpu.VMEM((8, _EMBED_DIM), jnp.float32),
            pltpu.SemaphoreType.DMA,
            pltpu.SemaphoreType.DMA,
            pltpu.SemaphoreType.DMA,
            pltpu.SemaphoreType.DMA,
            pltpu.SemaphoreType.DMA,
            pltpu.SemaphoreType.DMA,
            pltpu.SemaphoreType.DMA,
            pltpu.SemaphoreType.DMA,
        ],
    )(idx, weight)

    return lax.dynamic_update_slice(
        out_main, out_tail[:, :n_tail, :], (0, _T_MAIN, 0))
